# Initial kernel scaffold; baseline (speedup 1.0000x reference)
import jax
import jax.numpy as jnp
from jax.experimental import pallas as pl


def kernel(sequence, segment_label, emb, token_table, seg_table):
    raise NotImplementedError("write your pallas kernel here")


# speedup vs baseline: 3.9677x; 3.9677x over previous
import jax
import jax.numpy as jnp
import numpy as np
from jax.experimental import pallas as pl

_MAX_LEN = 512
_BB = 16


def _pe_buffer(max_len, d):
    pos = np.arange(max_len, dtype=np.float32)[:, None]
    i = np.arange(0, d, 2, dtype=np.float32)
    div = np.exp(-(np.log(10000.0)) * i / d)
    pe = np.zeros((max_len, d), dtype=np.float32)
    pe[:, 0::2] = np.sin(pos * div)
    pe[:, 1::2] = np.cos(pos * div)
    return pe


def _embed_block(seqt_ref, labt_ref, emb_ref, pe_ref, tok4_ref, seg_ref, out_ref):
    pe = pe_ref[...]
    tok4 = tok4_ref[...]
    seg = seg_ref[...]
    s0 = seg[0:1, :]
    s1 = seg[1:2, :]
    s2 = seg[2:3, :]
    for b in range(_BB):
        sb = seqt_ref[0, :, b : b + 1]
        lb = labt_ref[0, :, b : b + 1]
        e = emb_ref[b]
        y = jnp.where(sb == 4, tok4, e)
        s = jnp.where(lb == 0, s0, jnp.where(lb == 1, s1, s2))
        out_ref[b] = pe + y + s


def kernel(sequence, segment_label, emb, token_table, seg_table):
    b, l = sequence.shape
    d = token_table.shape[1]
    pe = jnp.asarray(_pe_buffer(_MAX_LEN, d)[:l])
    tok4 = jax.lax.slice(token_table, (4, 0), (5, d))
    nb = b // _BB
    seqt = sequence.reshape(nb, _BB, l).transpose(0, 2, 1)
    labt = segment_label.reshape(nb, _BB, l).transpose(0, 2, 1)

    grid = (nb,)
    out = pl.pallas_call(
        _embed_block,
        grid=grid,
        in_specs=[
            pl.BlockSpec((1, l, _BB), lambda i: (i, 0, 0)),
            pl.BlockSpec((1, l, _BB), lambda i: (i, 0, 0)),
            pl.BlockSpec((_BB, l, d), lambda i: (i, 0, 0)),
            pl.BlockSpec((l, d), lambda i: (0, 0)),
            pl.BlockSpec((1, d), lambda i: (0, 0)),
            pl.BlockSpec((3, d), lambda i: (0, 0)),
        ],
        out_specs=pl.BlockSpec((_BB, l, d), lambda i: (i, 0, 0)),
        out_shape=jax.ShapeDtypeStruct((b, l, d), jnp.float32),
    )(seqt, labt, emb, pe, tok4, seg_table)
    return out


# speedup vs baseline: 5.1113x; 1.2882x over previous
import jax
import jax.numpy as jnp
import numpy as np
from jax.experimental import pallas as pl

_MAX_LEN = 512
_BB = 16


def _pe_buffer(max_len, d):
    pos = np.arange(max_len, dtype=np.float32)[:, None]
    i = np.arange(0, d, 2, dtype=np.float32)
    div = np.exp(-(np.log(10000.0)) * i / d)
    pe = np.zeros((max_len, d), dtype=np.float32)
    pe[:, 0::2] = np.sin(pos * div)
    pe[:, 1::2] = np.cos(pos * div)
    return pe


def _embed_block(se_ref, so_ref, le_ref, lo_ref, emb_ref, pe_ref, tok4_ref,
                 seg_ref, out_ref):
    pe = pe_ref[...]
    tok4 = tok4_ref[...]
    seg = seg_ref[...]
    s0 = seg[0:1, :]
    s1 = seg[1:2, :]
    s2 = seg[2:3, :]
    h, w = pe.shape
    left = jax.lax.broadcasted_iota(jnp.int32, (h, w), 1) < (w // 2)
    for b in range(_BB):
        se = se_ref[0, :, b : b + 1]
        so = so_ref[0, :, b : b + 1]
        le = le_ref[0, :, b : b + 1]
        lo = lo_ref[0, :, b : b + 1]
        seq = jnp.where(left, se, so)
        lab = jnp.where(left, le, lo)
        e = emb_ref[b]
        y = jnp.where(seq == 4, tok4, e)
        s = jnp.where(lab == 0, s0, jnp.where(lab == 1, s1, s2))
        out_ref[b] = pe + y + s


def kernel(sequence, segment_label, emb, token_table, seg_table):
    b, l = sequence.shape
    d = token_table.shape[1]
    h = l // 2
    pe = jnp.asarray(_pe_buffer(_MAX_LEN, d)[:l].reshape(h, 2 * d))
    tok4 = jax.lax.slice(token_table, (4, 0), (5, d))
    tok4 = jnp.concatenate([tok4, tok4], axis=1)
    seg2 = jnp.concatenate([seg_table, seg_table], axis=1)

    nb = b // _BB
    def tile(ix):
        return ix.reshape(nb, _BB, h).transpose(0, 2, 1)

    se = tile(sequence[:, 0::2])
    so = tile(sequence[:, 1::2])
    le = tile(segment_label[:, 0::2])
    lo = tile(segment_label[:, 1::2])
    emb2 = emb.reshape(b, h, 2 * d)

    grid = (nb,)
    ispec = pl.BlockSpec((1, h, _BB), lambda i: (i, 0, 0))
    out = pl.pallas_call(
        _embed_block,
        grid=grid,
        in_specs=[
            ispec,
            ispec,
            ispec,
            ispec,
            pl.BlockSpec((_BB, h, 2 * d), lambda i: (i, 0, 0)),
            pl.BlockSpec((h, 2 * d), lambda i: (0, 0)),
            pl.BlockSpec((1, 2 * d), lambda i: (0, 0)),
            pl.BlockSpec((3, 2 * d), lambda i: (0, 0)),
        ],
        out_specs=pl.BlockSpec((_BB, h, 2 * d), lambda i: (i, 0, 0)),
        out_shape=jax.ShapeDtypeStruct((b, h, 2 * d), jnp.float32),
    )(se, so, le, lo, emb2, pe, tok4, seg2)
    return out.reshape(b, l, d)
